# baseline (speedup 1.0000x reference)
import functools

import jax
import jax.numpy as jnp
from jax import lax
from jax.experimental import pallas as pl
from jax.experimental.pallas import tpu as pltpu
from jax.experimental.pallas import tpu_sc as plsc

N_NODES = 10000
N_EDGES = 320000
D_FEAT = 128
D_EDGE = 16

NC = 2
NS = 16
NW = NC * NS
E_TILE = N_EDGES // NW
BATCH = 80
CHUNK = 2000
B_PER_CHUNK = CHUNK // BATCH
N_CHUNKS = E_TILE // CHUNK
N_PAD = 10240
N_SLICE = N_PAD // NS


def _seg_sum_body(j_hbm, ea_hbm, zeros_hbm, out_hbm,
                  idx_v, col_v, rows_v, acc_sh, sem_stage, sem_scat):
    c = lax.axis_index("c")
    s = lax.axis_index("s")
    wid = s * NC + c
    base_e = wid * E_TILE
    pltpu.sync_copy(zeros_hbm, acc_sh.at[pl.ds(s * N_SLICE, N_SLICE)])
    pltpu.sync_copy(j_hbm.at[pl.ds(base_e, E_TILE)], idx_v)
    plsc.subcore_barrier()

    lane16 = lax.broadcasted_iota(jnp.int32, (16,), 0)

    def stage(ch):
        for f in range(D_EDGE):
            pltpu.async_copy(
                ea_hbm.at[f].at[pl.ds(base_e + ch * CHUNK, CHUNK)],
                col_v.at[pl.ds(f * CHUNK, CHUNK)], sem_stage)

    def drain_stage():
        pltpu.make_async_copy(
            ea_hbm.at[0].at[pl.ds(0, D_EDGE * CHUNK)], col_v, sem_stage).wait()

    def drain_scat():
        pltpu.make_async_copy(
            out_hbm.at[0].at[pl.ds(0, CHUNK)],
            rows_v.at[pl.ds(0, CHUNK)], sem_scat).wait()

    stage(0)

    def chunk_body(ch, _):
        p = lax.rem(ch, 2)
        @pl.when(ch >= 2)
        def _():
            drain_scat()

        drain_stage()

        row0 = p * CHUNK

        def xpose(q, avec):
            rows_v[row0 + q] = plsc.load_gather(col_v, [avec])
            return avec + 1

        lax.fori_loop(0, CHUNK, xpose, lane16 * CHUNK, unroll=32)

        @pl.when(ch < N_CHUNKS - 1)
        def _():
            stage(ch + 1)

        def scat(b, _):
            pltpu.async_copy(
                rows_v.at[pl.ds(p * CHUNK + b * BATCH, BATCH)],
                acc_sh.at[idx_v.at[pl.ds(ch * CHUNK + b * BATCH, BATCH)]],
                sem_scat, add=True)
            return ()

        lax.fori_loop(0, B_PER_CHUNK, scat, (), unroll=True)
        return ()

    lax.fori_loop(0, N_CHUNKS, chunk_body, ())
    drain_scat()
    drain_scat()
    plsc.subcore_barrier()
    pltpu.sync_copy(
        acc_sh.at[pl.ds(s * N_SLICE, N_SLICE)],
        out_hbm.at[c].at[pl.ds(s * N_SLICE, N_SLICE)],
    )


@functools.cache
def _seg_sum():
    return pl.kernel(
        _seg_sum_body,
        out_type=jax.ShapeDtypeStruct((NC, N_PAD, D_EDGE), jnp.float32),
        mesh=plsc.VectorSubcoreMesh(
            core_axis_name="c", subcore_axis_name="s",
            num_cores=NC, num_subcores=NS,
        ),
        scratch_types=[
            pltpu.VMEM((E_TILE,), jnp.int32),
            pltpu.VMEM((D_EDGE * CHUNK,), jnp.float32),
            pltpu.VMEM((2 * CHUNK, D_EDGE), jnp.float32),
            pltpu.VMEM_SHARED((N_PAD, D_EDGE), jnp.float32),
            pltpu.SemaphoreType.DMA,
            pltpu.SemaphoreType.DMA,
        ],
        compiler_params=pltpu.CompilerParams(
            use_tc_tiling_on_sc=False, needs_layout_passes=False
        ),
    )


def _mlp_body(x_ref, p_ref, w1x_ref, w1a_ref, b1_ref, w2_ref, b2_ref,
              g_ref, bt_ref, o_ref):
    xb = x_ref[...]
    agg = p_ref[0] + p_ref[1]
    h = jnp.dot(xb, w1x_ref[...], preferred_element_type=jnp.float32)
    h = h + jnp.dot(agg, w1a_ref[...], preferred_element_type=jnp.float32)
    h = jnp.maximum(h + b1_ref[...], 0.0)
    h = jnp.dot(h, w2_ref[...], preferred_element_type=jnp.float32) + b2_ref[...]
    mean = jnp.mean(h, axis=1, keepdims=True)
    ctr = h - mean
    var = jnp.mean(ctr * ctr, axis=1, keepdims=True)
    o_ref[...] = xb + ctr * lax.rsqrt(var + 1e-5) * g_ref[...] + bt_ref[...]


_BLK = 1000


def _node_mlp(x, partials, w1x, w1a, b1, w2, b2, gamma, beta):
    nblk = N_NODES // _BLK
    full = lambda shape: pl.BlockSpec(shape, lambda i: (0,) * len(shape))
    return pl.pallas_call(
        _mlp_body,
        grid=(nblk,),
        in_specs=[
            pl.BlockSpec((_BLK, D_FEAT), lambda i: (i, 0)),
            pl.BlockSpec((NC, _BLK, D_EDGE), lambda i: (0, i, 0)),
            full((D_FEAT, 128)),
            full((D_EDGE, 128)),
            full((1, 128)),
            full((128, 128)),
            full((1, 128)),
            full((1, 128)),
            full((1, 128)),
        ],
        out_specs=pl.BlockSpec((_BLK, D_FEAT), lambda i: (i, 0)),
        out_shape=jax.ShapeDtypeStruct((N_NODES, D_FEAT), jnp.float32),
    )(x, partials, w1x, w1a, b1, w2, b2, gamma, beta)


def kernel(x, edge_index, edge_attr, W1, b1, W2, b2, gamma, beta):
    j = edge_index[1].astype(jnp.int32)
    ea_t = jnp.transpose(edge_attr)
    zeros = jnp.zeros((N_SLICE, D_EDGE), jnp.float32)
    partials = _seg_sum()(j, ea_t, zeros)
    return _node_mlp(
        x, partials, W1[:D_FEAT], W1[D_FEAT:], b1[None], W2, b2[None],
        gamma[None], beta[None],
    )


# speedup vs baseline: 1.1707x; 1.1707x over previous
import functools

import jax
import jax.numpy as jnp
from jax import lax
from jax.experimental import pallas as pl
from jax.experimental.pallas import tpu as pltpu
from jax.experimental.pallas import tpu_sc as plsc

N_NODES = 10000
N_EDGES = 320000
D_FEAT = 128
D_EDGE = 16

NC = 2
NS = 16
NW = NC * NS
E_TILE = N_EDGES // NW
BATCH = 80
CHUNK = 2000
B_PER_CHUNK = CHUNK // BATCH
N_CHUNKS = E_TILE // CHUNK
N_PAD = 10240
N_SLICE = N_PAD // NS


def _seg_sum_body(j_hbm, ea_hbm, zeros_hbm, out_hbm,
                  idx_v, col_v, rows_v, acc_sh, sem_stage, sem_scat):
    c = lax.axis_index("c")
    s = lax.axis_index("s")
    wid = s * NC + c
    base_e = wid * E_TILE
    pltpu.sync_copy(zeros_hbm, acc_sh.at[pl.ds(s * N_SLICE, N_SLICE)])
    pltpu.sync_copy(j_hbm.at[pl.ds(base_e, E_TILE)], idx_v)
    plsc.subcore_barrier()

    lane16 = lax.broadcasted_iota(jnp.int32, (16,), 0)

    def stage(ch):
        for f in range(D_EDGE):
            pltpu.async_copy(
                ea_hbm.at[f].at[pl.ds(base_e + ch * CHUNK, CHUNK)],
                col_v.at[pl.ds(f * CHUNK, CHUNK)], sem_stage)

    def drain_stage():
        pltpu.make_async_copy(
            ea_hbm.at[0].at[pl.ds(0, D_EDGE * CHUNK)], col_v, sem_stage).wait()

    def drain_scat():
        pltpu.make_async_copy(
            out_hbm.at[0].at[pl.ds(0, CHUNK)],
            rows_v.at[pl.ds(0, CHUNK)], sem_scat).wait()

    stage(0)

    def chunk_body(ch, _):
        p = lax.rem(ch, 2)
        @pl.when(ch >= 2)
        def _():
            drain_scat()

        drain_stage()

        for f in range(D_EDGE):
            f_vec = jnp.full((16,), f, jnp.int32)

            def xpose(q, e_vec, f=f, f_vec=f_vec):
                v = col_v[pl.ds(f * CHUNK + q * 16, 16)]
                plsc.store_scatter(rows_v, [e_vec, f_vec], v)
                return e_vec + 16

            lax.fori_loop(0, CHUNK // 16, xpose, lane16 + p * CHUNK, unroll=25)

        @pl.when(ch < N_CHUNKS - 1)
        def _():
            stage(ch + 1)

        def scat(b, _):
            pltpu.async_copy(
                rows_v.at[pl.ds(p * CHUNK + b * BATCH, BATCH)],
                acc_sh.at[idx_v.at[pl.ds(ch * CHUNK + b * BATCH, BATCH)]],
                sem_scat, add=True)
            return ()

        lax.fori_loop(0, B_PER_CHUNK, scat, (), unroll=True)
        return ()

    lax.fori_loop(0, N_CHUNKS, chunk_body, ())
    drain_scat()
    drain_scat()
    plsc.subcore_barrier()
    pltpu.sync_copy(
        acc_sh.at[pl.ds(s * N_SLICE, N_SLICE)],
        out_hbm.at[c].at[pl.ds(s * N_SLICE, N_SLICE)],
    )


@functools.cache
def _seg_sum():
    return pl.kernel(
        _seg_sum_body,
        out_type=jax.ShapeDtypeStruct((NC, N_PAD, D_EDGE), jnp.float32),
        mesh=plsc.VectorSubcoreMesh(
            core_axis_name="c", subcore_axis_name="s",
            num_cores=NC, num_subcores=NS,
        ),
        scratch_types=[
            pltpu.VMEM((E_TILE,), jnp.int32),
            pltpu.VMEM((D_EDGE * CHUNK,), jnp.float32),
            pltpu.VMEM((2 * CHUNK, D_EDGE), jnp.float32),
            pltpu.VMEM_SHARED((N_PAD, D_EDGE), jnp.float32),
            pltpu.SemaphoreType.DMA,
            pltpu.SemaphoreType.DMA,
        ],
        compiler_params=pltpu.CompilerParams(
            use_tc_tiling_on_sc=False, needs_layout_passes=False
        ),
    )


def _mlp_body(x_ref, p_ref, w1x_ref, w1a_ref, b1_ref, w2_ref, b2_ref,
              g_ref, bt_ref, o_ref):
    xb = x_ref[...]
    agg = p_ref[0] + p_ref[1]
    h = jnp.dot(xb, w1x_ref[...], preferred_element_type=jnp.float32)
    h = h + jnp.dot(agg, w1a_ref[...], preferred_element_type=jnp.float32)
    h = jnp.maximum(h + b1_ref[...], 0.0)
    h = jnp.dot(h, w2_ref[...], preferred_element_type=jnp.float32) + b2_ref[...]
    mean = jnp.mean(h, axis=1, keepdims=True)
    ctr = h - mean
    var = jnp.mean(ctr * ctr, axis=1, keepdims=True)
    o_ref[...] = xb + ctr * lax.rsqrt(var + 1e-5) * g_ref[...] + bt_ref[...]


_BLK = 1000


def _node_mlp(x, partials, w1x, w1a, b1, w2, b2, gamma, beta):
    nblk = N_NODES // _BLK
    full = lambda shape: pl.BlockSpec(shape, lambda i: (0,) * len(shape))
    return pl.pallas_call(
        _mlp_body,
        grid=(nblk,),
        in_specs=[
            pl.BlockSpec((_BLK, D_FEAT), lambda i: (i, 0)),
            pl.BlockSpec((NC, _BLK, D_EDGE), lambda i: (0, i, 0)),
            full((D_FEAT, 128)),
            full((D_EDGE, 128)),
            full((1, 128)),
            full((128, 128)),
            full((1, 128)),
            full((1, 128)),
            full((1, 128)),
        ],
        out_specs=pl.BlockSpec((_BLK, D_FEAT), lambda i: (i, 0)),
        out_shape=jax.ShapeDtypeStruct((N_NODES, D_FEAT), jnp.float32),
    )(x, partials, w1x, w1a, b1, w2, b2, gamma, beta)


def kernel(x, edge_index, edge_attr, W1, b1, W2, b2, gamma, beta):
    j = edge_index[1].astype(jnp.int32)
    ea_t = jnp.transpose(edge_attr)
    zeros = jnp.zeros((N_SLICE, D_EDGE), jnp.float32)
    partials = _seg_sum()(j, ea_t, zeros)
    return _node_mlp(
        x, partials, W1[:D_FEAT], W1[D_FEAT:], b1[None], W2, b2[None],
        gamma[None], beta[None],
    )


# speedup vs baseline: 1.2625x; 1.0785x over previous
import functools

import jax
import jax.numpy as jnp
from jax import lax
from jax.experimental import pallas as pl
from jax.experimental.pallas import tpu as pltpu
from jax.experimental.pallas import tpu_sc as plsc

N_NODES = 10000
N_EDGES = 320000
D_FEAT = 128
D_EDGE = 16

NC = 2
NS = 16
NW = NC * NS
E_TILE = N_EDGES // NW
BATCH = 2000
CHUNK = 2000
B_PER_CHUNK = CHUNK // BATCH
N_CHUNKS = E_TILE // CHUNK
N_PAD = 10240
N_SLICE = N_PAD // NS


def _seg_sum_body(j_hbm, ea_hbm, zeros_hbm, out_hbm,
                  idx_v, col_v, rows_v, acc_sh, sem_stage, sem_scat):
    c = lax.axis_index("c")
    s = lax.axis_index("s")
    wid = s * NC + c
    base_e = wid * E_TILE
    pltpu.sync_copy(zeros_hbm, acc_sh.at[pl.ds(s * N_SLICE, N_SLICE)])
    pltpu.sync_copy(j_hbm.at[pl.ds(base_e, E_TILE)], idx_v)
    plsc.subcore_barrier()

    lane16 = lax.broadcasted_iota(jnp.int32, (16,), 0)

    def stage(ch):
        for f in range(D_EDGE):
            pltpu.async_copy(
                ea_hbm.at[f].at[pl.ds(base_e + ch * CHUNK, CHUNK)],
                col_v.at[pl.ds(f * CHUNK, CHUNK)], sem_stage)

    def drain_stage():
        pltpu.make_async_copy(
            ea_hbm.at[0].at[pl.ds(0, D_EDGE * CHUNK)], col_v, sem_stage).wait()

    def drain_scat():
        pltpu.make_async_copy(
            out_hbm.at[0].at[pl.ds(0, CHUNK)],
            rows_v.at[pl.ds(0, CHUNK)], sem_scat).wait()

    stage(0)

    def chunk_body(ch, _):
        p = lax.rem(ch, 2)
        @pl.when(ch >= 2)
        def _():
            drain_scat()

        drain_stage()

        for r in range(D_EDGE):
            fr = lax.rem(lane16 + r, D_EDGE)
            g0 = fr * CHUNK + lane16

            def xpose(q, carry, fr=fr):
                g_vec, e_vec = carry
                v = plsc.load_gather(col_v, [g_vec])
                plsc.store_scatter(rows_v, [e_vec, fr], v)
                return g_vec + 16, e_vec + 16

            lax.fori_loop(0, CHUNK // 16, xpose,
                          (g0, lane16 + p * CHUNK), unroll=4)

        @pl.when(ch < N_CHUNKS - 1)
        def _():
            stage(ch + 1)

        def scat(b, _):
            pltpu.async_copy(
                rows_v.at[pl.ds(p * CHUNK + b * BATCH, BATCH)],
                acc_sh.at[idx_v.at[pl.ds(ch * CHUNK + b * BATCH, BATCH)]],
                sem_scat, add=True)
            return ()

        lax.fori_loop(0, B_PER_CHUNK, scat, (), unroll=True)
        return ()

    lax.fori_loop(0, N_CHUNKS, chunk_body, ())
    drain_scat()
    drain_scat()
    plsc.subcore_barrier()
    pltpu.sync_copy(
        acc_sh.at[pl.ds(s * N_SLICE, N_SLICE)],
        out_hbm.at[c].at[pl.ds(s * N_SLICE, N_SLICE)],
    )


@functools.cache
def _seg_sum():
    return pl.kernel(
        _seg_sum_body,
        out_type=jax.ShapeDtypeStruct((NC, N_PAD, D_EDGE), jnp.float32),
        mesh=plsc.VectorSubcoreMesh(
            core_axis_name="c", subcore_axis_name="s",
            num_cores=NC, num_subcores=NS,
        ),
        scratch_types=[
            pltpu.VMEM((E_TILE,), jnp.int32),
            pltpu.VMEM((D_EDGE * CHUNK,), jnp.float32),
            pltpu.VMEM((2 * CHUNK, D_EDGE), jnp.float32),
            pltpu.VMEM_SHARED((N_PAD, D_EDGE), jnp.float32),
            pltpu.SemaphoreType.DMA,
            pltpu.SemaphoreType.DMA,
        ],
        compiler_params=pltpu.CompilerParams(
            use_tc_tiling_on_sc=False, needs_layout_passes=False
        ),
    )


def _mlp_body(x_ref, p_ref, w1x_ref, w1a_ref, b1_ref, w2_ref, b2_ref,
              g_ref, bt_ref, o_ref):
    xb = x_ref[...]
    agg = p_ref[0] + p_ref[1]
    h = jnp.dot(xb, w1x_ref[...], preferred_element_type=jnp.float32)
    h = h + jnp.dot(agg, w1a_ref[...], preferred_element_type=jnp.float32)
    h = jnp.maximum(h + b1_ref[...], 0.0)
    h = jnp.dot(h, w2_ref[...], preferred_element_type=jnp.float32) + b2_ref[...]
    mean = jnp.mean(h, axis=1, keepdims=True)
    ctr = h - mean
    var = jnp.mean(ctr * ctr, axis=1, keepdims=True)
    o_ref[...] = xb + ctr * lax.rsqrt(var + 1e-5) * g_ref[...] + bt_ref[...]


_BLK = 1000


def _node_mlp(x, partials, w1x, w1a, b1, w2, b2, gamma, beta):
    nblk = N_NODES // _BLK
    full = lambda shape: pl.BlockSpec(shape, lambda i: (0,) * len(shape))
    return pl.pallas_call(
        _mlp_body,
        grid=(nblk,),
        in_specs=[
            pl.BlockSpec((_BLK, D_FEAT), lambda i: (i, 0)),
            pl.BlockSpec((NC, _BLK, D_EDGE), lambda i: (0, i, 0)),
            full((D_FEAT, 128)),
            full((D_EDGE, 128)),
            full((1, 128)),
            full((128, 128)),
            full((1, 128)),
            full((1, 128)),
            full((1, 128)),
        ],
        out_specs=pl.BlockSpec((_BLK, D_FEAT), lambda i: (i, 0)),
        out_shape=jax.ShapeDtypeStruct((N_NODES, D_FEAT), jnp.float32),
    )(x, partials, w1x, w1a, b1, w2, b2, gamma, beta)


def kernel(x, edge_index, edge_attr, W1, b1, W2, b2, gamma, beta):
    j = edge_index[1].astype(jnp.int32)
    ea_t = jnp.transpose(edge_attr)
    zeros = jnp.zeros((N_SLICE, D_EDGE), jnp.float32)
    partials = _seg_sum()(j, ea_t, zeros)
    return _node_mlp(
        x, partials, W1[:D_FEAT], W1[D_FEAT:], b1[None], W2, b2[None],
        gamma[None], beta[None],
    )


# speedup vs baseline: 1.3655x; 1.0815x over previous
import functools

import jax
import jax.numpy as jnp
from jax import lax
from jax.experimental import pallas as pl
from jax.experimental.pallas import tpu as pltpu
from jax.experimental.pallas import tpu_sc as plsc

N_NODES = 10000
N_EDGES = 320000
D_FEAT = 128
D_EDGE = 16

NC = 2
NS = 16
NW = NC * NS
E_TILE = N_EDGES // NW
BATCH = 2000
CHUNK = 2000
B_PER_CHUNK = CHUNK // BATCH
N_CHUNKS = E_TILE // CHUNK
N_PAD = 10240
N_SLICE = N_PAD // NS


def _seg_sum_body(j_hbm, ea_hbm, zeros_hbm, out_hbm,
                  idx_v, col_v, rows_v, acc_sh, sem_stage, sem_scat):
    c = lax.axis_index("c")
    s = lax.axis_index("s")
    wid = s * NC + c
    base_e = wid * E_TILE

    lane16 = lax.broadcasted_iota(jnp.int32, (16,), 0)

    def stage(ch):
        for f in range(D_EDGE):
            pltpu.async_copy(
                ea_hbm.at[f].at[pl.ds(base_e + ch * CHUNK, CHUNK)],
                col_v.at[pl.ds(f * CHUNK, CHUNK)], sem_stage)

    def drain_stage():
        pltpu.make_async_copy(
            ea_hbm.at[0].at[pl.ds(0, D_EDGE * CHUNK)], col_v, sem_stage).wait()

    def drain_scat():
        pltpu.make_async_copy(
            out_hbm.at[0].at[pl.ds(0, CHUNK)],
            rows_v.at[pl.ds(0, CHUNK)], sem_scat).wait()

    stage(0)
    pltpu.sync_copy(zeros_hbm, acc_sh.at[pl.ds(s * N_SLICE, N_SLICE)])
    pltpu.sync_copy(j_hbm.at[pl.ds(base_e, E_TILE)], idx_v)
    plsc.subcore_barrier()

    def chunk_body(ch, _):
        p = lax.rem(ch, 2)
        @pl.when(ch >= 2)
        def _():
            drain_scat()

        drain_stage()

        for r in range(D_EDGE):
            fr = lax.rem(lane16 + r, D_EDGE)
            g0 = fr * CHUNK + lane16

            def xpose(q, carry, fr=fr):
                g_vec, e_vec = carry
                v = plsc.load_gather(col_v, [g_vec])
                plsc.store_scatter(rows_v, [e_vec, fr], v)
                return g_vec + 16, e_vec + 16

            lax.fori_loop(0, CHUNK // 16, xpose,
                          (g0, lane16 + p * CHUNK), unroll=4)

        @pl.when(ch < N_CHUNKS - 1)
        def _():
            stage(ch + 1)

        def scat(b, _):
            pltpu.async_copy(
                rows_v.at[pl.ds(p * CHUNK + b * BATCH, BATCH)],
                acc_sh.at[idx_v.at[pl.ds(ch * CHUNK + b * BATCH, BATCH)]],
                sem_scat, add=True)
            return ()

        lax.fori_loop(0, B_PER_CHUNK, scat, (), unroll=True)
        return ()

    lax.fori_loop(0, N_CHUNKS, chunk_body, ())
    drain_scat()
    drain_scat()
    plsc.subcore_barrier()
    pltpu.sync_copy(
        acc_sh.at[pl.ds(s * N_SLICE, N_SLICE)],
        out_hbm.at[c].at[pl.ds(s * N_SLICE, N_SLICE)],
    )


@functools.cache
def _seg_sum():
    return pl.kernel(
        _seg_sum_body,
        out_type=jax.ShapeDtypeStruct((NC, N_PAD, D_EDGE), jnp.float32),
        mesh=plsc.VectorSubcoreMesh(
            core_axis_name="c", subcore_axis_name="s",
            num_cores=NC, num_subcores=NS,
        ),
        scratch_types=[
            pltpu.VMEM((E_TILE,), jnp.int32),
            pltpu.VMEM((D_EDGE * CHUNK,), jnp.float32),
            pltpu.VMEM((2 * CHUNK, D_EDGE), jnp.float32),
            pltpu.VMEM_SHARED((N_PAD, D_EDGE), jnp.float32),
            pltpu.SemaphoreType.DMA,
            pltpu.SemaphoreType.DMA,
        ],
        compiler_params=pltpu.CompilerParams(
            use_tc_tiling_on_sc=False, needs_layout_passes=False
        ),
    )


def _mlp_body(x_ref, p_ref, w1x_ref, bd_ref, b1_ref, w2_ref, b2_ref,
              g_ref, bt_ref, o_ref):
    xb = x_ref[...]
    psum = p_ref[0] + p_ref[1]
    aggw = jnp.dot(psum, bd_ref[...],
                   preferred_element_type=jnp.float32)
    h = jnp.dot(xb, w1x_ref[...], preferred_element_type=jnp.float32)
    h = h + aggw.reshape(_BLK, 128)
    h = jnp.maximum(h + b1_ref[...], 0.0)
    h = jnp.dot(h, w2_ref[...], preferred_element_type=jnp.float32) + b2_ref[...]
    mean = jnp.mean(h, axis=1, keepdims=True)
    ctr = h - mean
    var = jnp.mean(ctr * ctr, axis=1, keepdims=True)
    o_ref[...] = xb + ctr * lax.rsqrt(var + 1e-5) * g_ref[...] + bt_ref[...]


_BLK = 1024


def _node_mlp(x, p128, w1x, bd, b1, w2, b2, gamma, beta):
    nblk = (N_NODES + _BLK - 1) // _BLK
    full = lambda shape: pl.BlockSpec(shape, lambda i: (0,) * len(shape))
    return pl.pallas_call(
        _mlp_body,
        grid=(nblk,),
        in_specs=[
            pl.BlockSpec((_BLK, D_FEAT), lambda i: (i, 0)),
            pl.BlockSpec((NC, _BLK // 8, 128), lambda i: (0, i, 0)),
            full((D_FEAT, 128)),
            full((128, 8 * 128)),
            full((1, 128)),
            full((128, 128)),
            full((1, 128)),
            full((1, 128)),
            full((1, 128)),
        ],
        out_specs=pl.BlockSpec((_BLK, D_FEAT), lambda i: (i, 0)),
        out_shape=jax.ShapeDtypeStruct((N_NODES, D_FEAT), jnp.float32),
    )(x, p128, w1x, bd, b1, w2, b2, gamma, beta)


def kernel(x, edge_index, edge_attr, W1, b1, W2, b2, gamma, beta):
    j = edge_index[1].astype(jnp.int32)
    ea_t = jnp.transpose(edge_attr)
    zeros = jnp.zeros((N_SLICE, D_EDGE), jnp.float32)
    partials = _seg_sum()(j, ea_t, zeros)
    p128 = partials.reshape(NC, N_PAD * D_EDGE // 128, 128)
    bd = jnp.kron(jnp.eye(8, dtype=jnp.float32), W1[D_FEAT:])
    return _node_mlp(
        x, p128, W1[:D_FEAT], bd, b1[None], W2, b2[None],
        gamma[None], beta[None],
    )
